# baseline (speedup 1.0000x reference)
import functools

import jax
import jax.numpy as jnp
from jax import lax
from jax.experimental import pallas as pl
from jax.experimental.pallas import tpu as pltpu

N = 1024
PACK = 16
NJ = N // PACK
IBLK = 64
GRID = N // IBLK


def _sweep_kernel(e_ref, v_ref, vp_ref, u_ref,
                  wee_ref, wer_ref, wes_ref, weu_ref, bet_ref,
                  wnv_ref, wne_ref, wnu_ref, bn_ref,
                  wgu_ref, wgv_ref, wge_ref, bg_ref,
                  eo_ref, vo_ref, uo_ref,
                  rrep_scr, spack_scr, cpack_scr, agg_scr,
                  *, act_relu, residual, kin, kout):
    b = pl.program_id(0)
    prec = lax.Precision.HIGHEST
    e_out = kout // PACK

    @pl.when(b == 0)
    def _prologue():
        rrep_scr[...] = jnp.dot(v_ref[...], wer_ref[...], precision=prec)
        spack_scr[...] = jnp.dot(vp_ref[...], wes_ref[...], precision=prec)
        cpack_scr[...] = jnp.dot(u_ref[...], weu_ref[...], precision=prec) + bet_ref[...]

    x = e_ref[...]
    x2 = x.reshape(IBLK * NJ, kin)
    y2 = jnp.dot(x2, wee_ref[...], precision=prec)
    y = y2.reshape(IBLK, NJ, kout)
    r = rrep_scr[pl.ds(b * IBLK, IBLK), :]
    z = y + r[:, None, :] + spack_scr[...][None, :, :] + cpack_scr[...][None, :, :]
    if act_relu:
        z = jnp.maximum(z, 0.0)
    agg_scr[pl.ds(b * IBLK, IBLK), :] = z.sum(axis=1)
    if residual:
        eo_ref[...] = x + z
    else:
        eo_ref[...] = z

    @pl.when(b == GRID - 1)
    def _epilogue():
        aggp = agg_scr[...]
        agg = aggp.reshape(N, PACK, e_out).sum(axis=1) / float(N)
        esum = jnp.sum(aggp.reshape(N, PACK, e_out), axis=(0, 1)) / float(N * N)
        v = v_ref[...]
        u = u_ref[...]
        dv = (jnp.dot(v, wnv_ref[...], precision=prec)
              + jnp.dot(agg, wne_ref[...], precision=prec)
              + jnp.dot(u, wnu_ref[...], precision=prec)
              + bn_ref[...])
        if act_relu:
            dv = jnp.maximum(dv, 0.0)
        vmean = jnp.mean(dv, axis=0, keepdims=True)
        du = (jnp.dot(u, wgu_ref[...], precision=prec)
              + jnp.dot(vmean, wgv_ref[...], precision=prec)
              + jnp.dot(esum[None, :], wge_ref[...], precision=prec)
              + bg_ref[...])
        if act_relu:
            du = jnp.maximum(du, 0.0)
        if residual:
            vo_ref[...] = v + dv
            uo_ref[...] = u + du
        else:
            vo_ref[...] = dv
            uo_ref[...] = du


def _gn_sweep(E, V, u, wp, *, act_relu, residual):
    kin = E.shape[-1]
    kout = wp['Wee'].shape[-1]
    n_in = V.shape[-1]
    n_out = wp['Wnv'].shape[-1]
    g_out = wp['Wgu'].shape[-1]
    Vp = V.reshape(NJ, PACK * n_in)

    kfn = functools.partial(_sweep_kernel, act_relu=act_relu,
                            residual=residual, kin=kin, kout=kout)
    full = lambda shp: pl.BlockSpec(shp, lambda b: (0,) * len(shp))
    eo, vo, uo = pl.pallas_call(
        kfn,
        grid=(GRID,),
        in_specs=[
            pl.BlockSpec((IBLK, NJ, kin), lambda b: (b, 0, 0)),
            full((N, n_in)),
            full((NJ, PACK * n_in)),
            full((1, u.shape[-1])),
            full(wp['Wee'].shape),
            full(wp['Wer'].shape),
            full(wp['Wes'].shape),
            full(wp['Weu'].shape),
            full(wp['bet'].shape),
            full(wp['Wnv'].shape),
            full(wp['Wne'].shape),
            full(wp['Wnu'].shape),
            full(wp['bn'].shape),
            full(wp['Wgu'].shape),
            full(wp['Wgv'].shape),
            full(wp['Wge'].shape),
            full(wp['bg'].shape),
        ],
        out_specs=[
            pl.BlockSpec((IBLK, NJ, kout), lambda b: (b, 0, 0)),
            full((N, n_out)),
            full((1, g_out)),
        ],
        out_shape=[
            jax.ShapeDtypeStruct((N, NJ, kout), jnp.float32),
            jax.ShapeDtypeStruct((N, n_out), jnp.float32),
            jax.ShapeDtypeStruct((1, g_out), jnp.float32),
        ],
        scratch_shapes=[
            pltpu.VMEM((N, kout), jnp.float32),
            pltpu.VMEM((NJ, kout), jnp.float32),
            pltpu.VMEM((1, kout), jnp.float32),
            pltpu.VMEM((N, kout), jnp.float32),
        ],
        compiler_params=pltpu.CompilerParams(
            dimension_semantics=("arbitrary",)),
    )(E, V, Vp, u,
      wp['Wee'], wp['Wer'], wp['Wes'], wp['Weu'], wp['bet'],
      wp['Wnv'], wp['Wne'], wp['Wnu'], wp['bn'],
      wp['Wgu'], wp['Wgv'], wp['Wge'], wp['bg'])
    return eo, vo, uo


def _prep_block(p):
    e_in, e_out = p['We_e'].shape
    eye = jnp.eye(PACK, dtype=jnp.float32)
    return {
        'Wee': jnp.kron(eye, p['We_e']),
        'Wer': jnp.tile(p['We_r'], (1, PACK)),
        'Wes': jnp.kron(eye, p['We_s']),
        'Weu': jnp.tile(p['We_u'], (1, PACK)),
        'bet': jnp.tile(p['be'], PACK)[None, :],
        'Wnv': p['Wn_v'], 'Wne': p['Wn_e'], 'Wnu': p['Wn_u'],
        'bn': p['bn'][None, :],
        'Wgu': p['Wg_u'], 'Wgv': p['Wg_v'], 'Wge': p['Wg_e'],
        'bg': p['bg'][None, :],
    }


def kernel(u, V, A, params):
    e_in = A.shape[-1]
    E = A.reshape(N, NJ, PACK * e_in)
    uc = u[None, :]
    E, V, uc = _gn_sweep(E, V, uc, _prep_block(params['enc']),
                         act_relu=True, residual=False)
    for p in params['proc']:
        E, V, uc = _gn_sweep(E, V, uc, _prep_block(p),
                             act_relu=True, residual=True)
    E, V, uc = _gn_sweep(E, V, uc, _prep_block(params['dec']),
                         act_relu=False, residual=False)
    e_out = params['dec']['We_e'].shape[-1]
    return uc[0], V, E.reshape(N, N, e_out)


# speedup vs baseline: 1.4859x; 1.4859x over previous
import functools

import jax
import jax.numpy as jnp
from jax import lax
from jax.experimental import pallas as pl
from jax.experimental.pallas import tpu as pltpu

N = 1024
PACK = 16
NJ = N // PACK
IBLK = 64
GRID = N // IBLK


def _sweep_kernel(e_ref, v_ref, vp_ref, u_ref,
                  wee_ref, wer_ref, wes_ref, weu_ref, bet_ref,
                  wnv_ref, wne_ref, wnu_ref, bn_ref,
                  wgu_ref, wgv_ref, wge_ref, bg_ref,
                  eo_ref, vo_ref, uo_ref,
                  rrep_scr, spack_scr, cpack_scr, agg_scr,
                  *, act_relu, residual, kin, kout):
    b = pl.program_id(0)
    prec = lax.Precision.HIGHEST
    e_out = kout // PACK
    out_dtype = eo_ref.dtype

    @pl.when(b == 0)
    def _prologue():
        rrep_scr[...] = jnp.dot(v_ref[...], wer_ref[...], precision=prec)
        spack_scr[...] = jnp.dot(vp_ref[...], wes_ref[...], precision=prec)
        cpack_scr[...] = jnp.dot(u_ref[...], weu_ref[...], precision=prec) + bet_ref[...]

    x = e_ref[...]
    x2 = x.reshape(IBLK * NJ, kin)
    y2 = jnp.dot(x2, wee_ref[...],
                 preferred_element_type=jnp.float32)
    y = y2.reshape(IBLK, NJ, kout)
    r = rrep_scr[pl.ds(b * IBLK, IBLK), :]
    z = y + r[:, None, :] + spack_scr[...][None, :, :] + cpack_scr[...][None, :, :]
    if act_relu:
        z = jnp.maximum(z, 0.0)
    agg_scr[pl.ds(b * IBLK, IBLK), :] = z.sum(axis=1)
    if residual:
        eo_ref[...] = (x.astype(jnp.float32) + z).astype(out_dtype)
    else:
        eo_ref[...] = z.astype(out_dtype)

    @pl.when(b == GRID - 1)
    def _epilogue():
        aggp = agg_scr[...]
        agg = aggp.reshape(N, PACK, e_out).sum(axis=1) / float(N)
        esum = jnp.sum(aggp.reshape(N, PACK, e_out), axis=(0, 1)) / float(N * N)
        v = v_ref[...]
        u = u_ref[...]
        dv = (jnp.dot(v, wnv_ref[...], precision=prec)
              + jnp.dot(agg, wne_ref[...], precision=prec)
              + jnp.dot(u, wnu_ref[...], precision=prec)
              + bn_ref[...])
        if act_relu:
            dv = jnp.maximum(dv, 0.0)
        vmean = jnp.mean(dv, axis=0, keepdims=True)
        du = (jnp.dot(u, wgu_ref[...], precision=prec)
              + jnp.dot(vmean, wgv_ref[...], precision=prec)
              + jnp.dot(esum[None, :], wge_ref[...], precision=prec)
              + bg_ref[...])
        if act_relu:
            du = jnp.maximum(du, 0.0)
        if residual:
            vo_ref[...] = v + dv
            uo_ref[...] = u + du
        else:
            vo_ref[...] = dv
            uo_ref[...] = du


def _gn_sweep(E, V, u, wp, *, act_relu, residual, e_dtype=jnp.float32):
    kin = E.shape[-1]
    kout = wp['Wee'].shape[-1]
    n_in = V.shape[-1]
    n_out = wp['Wnv'].shape[-1]
    g_out = wp['Wgu'].shape[-1]
    Vp = V.reshape(NJ, PACK * n_in)

    kfn = functools.partial(_sweep_kernel, act_relu=act_relu,
                            residual=residual, kin=kin, kout=kout)
    full = lambda shp: pl.BlockSpec(shp, lambda b: (0,) * len(shp))
    eo, vo, uo = pl.pallas_call(
        kfn,
        grid=(GRID,),
        in_specs=[
            pl.BlockSpec((IBLK, NJ, kin), lambda b: (b, 0, 0)),
            full((N, n_in)),
            full((NJ, PACK * n_in)),
            full((1, u.shape[-1])),
            full(wp['Wee'].shape),
            full(wp['Wer'].shape),
            full(wp['Wes'].shape),
            full(wp['Weu'].shape),
            full(wp['bet'].shape),
            full(wp['Wnv'].shape),
            full(wp['Wne'].shape),
            full(wp['Wnu'].shape),
            full(wp['bn'].shape),
            full(wp['Wgu'].shape),
            full(wp['Wgv'].shape),
            full(wp['Wge'].shape),
            full(wp['bg'].shape),
        ],
        out_specs=[
            pl.BlockSpec((IBLK, NJ, kout), lambda b: (b, 0, 0)),
            full((N, n_out)),
            full((1, g_out)),
        ],
        out_shape=[
            jax.ShapeDtypeStruct((N, NJ, kout), e_dtype),
            jax.ShapeDtypeStruct((N, n_out), jnp.float32),
            jax.ShapeDtypeStruct((1, g_out), jnp.float32),
        ],
        scratch_shapes=[
            pltpu.VMEM((N, kout), jnp.float32),
            pltpu.VMEM((NJ, kout), jnp.float32),
            pltpu.VMEM((1, kout), jnp.float32),
            pltpu.VMEM((N, kout), jnp.float32),
        ],
        compiler_params=pltpu.CompilerParams(
            dimension_semantics=("arbitrary",)),
    )(E, V, Vp, u,
      wp['Wee'], wp['Wer'], wp['Wes'], wp['Weu'], wp['bet'],
      wp['Wnv'], wp['Wne'], wp['Wnu'], wp['bn'],
      wp['Wgu'], wp['Wgv'], wp['Wge'], wp['bg'])
    return eo, vo, uo


def _prep_block(p):
    e_in, e_out = p['We_e'].shape
    eye = jnp.eye(PACK, dtype=jnp.float32)
    return {
        'Wee': jnp.kron(eye, p['We_e']),
        'Wer': jnp.tile(p['We_r'], (1, PACK)),
        'Wes': jnp.kron(eye, p['We_s']),
        'Weu': jnp.tile(p['We_u'], (1, PACK)),
        'bet': jnp.tile(p['be'], PACK)[None, :],
        'Wnv': p['Wn_v'], 'Wne': p['Wn_e'], 'Wnu': p['Wn_u'],
        'bn': p['bn'][None, :],
        'Wgu': p['Wg_u'], 'Wgv': p['Wg_v'], 'Wge': p['Wg_e'],
        'bg': p['bg'][None, :],
    }


def kernel(u, V, A, params):
    e_in = A.shape[-1]
    E = A.reshape(N, NJ, PACK * e_in)
    uc = u[None, :]
    E, V, uc = _gn_sweep(E, V, uc, _prep_block(params['enc']),
                         act_relu=True, residual=False, e_dtype=jnp.bfloat16)
    for p in params['proc']:
        E, V, uc = _gn_sweep(E, V, uc, _prep_block(p),
                             act_relu=True, residual=True, e_dtype=jnp.bfloat16)
    E, V, uc = _gn_sweep(E, V, uc, _prep_block(params['dec']),
                         act_relu=False, residual=False)
    e_out = params['dec']['We_e'].shape[-1]
    return uc[0], V, E.reshape(N, N, e_out)
